# baseline (speedup 1.0000x reference)
import functools

import jax
import jax.numpy as jnp
from jax import lax
from jax.experimental import pallas as pl
from jax.experimental.pallas import tpu as pltpu
from jax.experimental.pallas import tpu_sc as plsc

VOCAB = 100000
EMBED = 64
OUT_DIM = 2
SEQ = 200
BATCH = 4096

NC = 2
NS = 16
NW = NC * NS
LANES = 16
BPT = BATCH // NW

VBLK = 20000
VGRID = VOCAB // VBLK


def _bf16_bits(p):
    u = lax.bitcast_convert_type(p, jnp.uint32)
    return (u + jnp.uint32(0x7FFF) + ((u >> 16) & jnp.uint32(1))) >> 16


def _fold_body(table_ref, wt_ref, b_ref, out_ref):
    t = table_ref[...]
    w = wt_ref[...]
    p = lax.dot_general(w, t, (((1,), (1,)), ((), ())),
                        preferred_element_type=jnp.float32)
    p0 = p[0:1, :] + b_ref[0]
    p1 = p[1:2, :] + b_ref[1]
    packed = _bf16_bits(p0) | (_bf16_bits(p1) << 16)
    out_ref[...] = lax.bitcast_convert_type(packed, jnp.int32).reshape(out_ref.shape)


def _fold_table(table, wt_pad, b):
    out = pl.pallas_call(
        _fold_body,
        grid=(VGRID,),
        in_specs=[
            pl.BlockSpec((VBLK, EMBED), lambda g: (g, 0)),
            pl.BlockSpec((8, EMBED), lambda g: (0, 0)),
            pl.BlockSpec(memory_space=pltpu.SMEM),
        ],
        out_specs=pl.BlockSpec((1, 1, VBLK), lambda g: (g, 0, 0)),
        out_shape=jax.ShapeDtypeStruct((VGRID, 1, VBLK), jnp.int32),
    )(table, wt_pad, b)
    return out.reshape(VOCAB)


def _sc_body(ptab_hbm, text_hbm, out_hbm, tab_v, idx_v, out_v):
    wid = lax.axis_index("s") * NC + lax.axis_index("c")
    base = wid * BPT
    pltpu.sync_copy(ptab_hbm, tab_v)
    pltpu.sync_copy(text_hbm.at[:, pl.ds(base, BPT)], idx_v)
    scale = jnp.float32(1.0 / SEQ)
    for bg in range(BPT // LANES):
        def body(s, acc, _bg=bg):
            a0, a1 = acc
            vocab = idx_v[s, pl.ds(_bg * LANES, LANES)]
            packed = plsc.load_gather(tab_v, [vocab])
            c0 = plsc.bitcast(packed << 16, jnp.float32)
            c1 = plsc.bitcast(packed & jnp.int32(-65536), jnp.float32)
            return (a0 + c0, a1 + c1)
        z = jnp.zeros((LANES,), jnp.float32)
        a0, a1 = lax.fori_loop(0, SEQ, body, (z, z), unroll=25)
        out_v[0, pl.ds(bg * LANES, LANES)] = a0 * scale
        out_v[1, pl.ds(bg * LANES, LANES)] = a1 * scale
    pltpu.sync_copy(out_v, out_hbm.at[:, pl.ds(base, BPT)])


@functools.lru_cache(maxsize=1)
def _sc_pool():
    return pl.kernel(
        _sc_body,
        out_type=jax.ShapeDtypeStruct((OUT_DIM, BATCH), jnp.float32),
        mesh=plsc.VectorSubcoreMesh(
            core_axis_name="c", subcore_axis_name="s", num_cores=NC, num_subcores=NS
        ),
        scratch_types=[
            pltpu.VMEM((VOCAB,), jnp.int32),
            pltpu.VMEM((SEQ, BPT), jnp.int32),
            pltpu.VMEM((OUT_DIM, BPT), jnp.float32),
        ],
        compiler_params=pltpu.CompilerParams(needs_layout_passes=False),
    )


def kernel(text, table, W, b):
    wt_pad = jnp.zeros((8, EMBED), jnp.float32).at[:OUT_DIM].set(W.T)
    ptab = _fold_table(table, wt_pad, b)
    return _sc_pool()(ptab, text).T


# speedup vs baseline: 1.0202x; 1.0202x over previous
import functools

import jax
import jax.numpy as jnp
from jax import lax
from jax.experimental import pallas as pl
from jax.experimental.pallas import tpu as pltpu
from jax.experimental.pallas import tpu_sc as plsc

VOCAB = 100000
EMBED = 64
OUT_DIM = 2
SEQ = 200
BATCH = 4096

NC = 2
NS = 16
NW = NC * NS
LANES = 16
BPT = BATCH // NW

VBLK = 20000
VGRID = VOCAB // VBLK


def _bf16_bits(p):
    u = lax.bitcast_convert_type(p, jnp.uint32)
    return (u + jnp.uint32(0x7FFF) + ((u >> 16) & jnp.uint32(1))) >> 16


def _fold_body(table_ref, wt_ref, b_ref, out_ref):
    t = table_ref[...]
    w = wt_ref[...]
    p = lax.dot_general(w, t, (((1,), (1,)), ((), ())),
                        preferred_element_type=jnp.float32)
    p0 = p[0:1, :] + b_ref[0]
    p1 = p[1:2, :] + b_ref[1]
    packed = _bf16_bits(p0) | (_bf16_bits(p1) << 16)
    out_ref[...] = lax.bitcast_convert_type(packed, jnp.int32).reshape(out_ref.shape)


def _fold_table(table, wt_pad, b):
    out = pl.pallas_call(
        _fold_body,
        grid=(VGRID,),
        in_specs=[
            pl.BlockSpec((VBLK, EMBED), lambda g: (g, 0)),
            pl.BlockSpec((8, EMBED), lambda g: (0, 0)),
            pl.BlockSpec(memory_space=pltpu.SMEM),
        ],
        out_specs=pl.BlockSpec((1, 1, VBLK), lambda g: (g, 0, 0)),
        out_shape=jax.ShapeDtypeStruct((VGRID, 1, VBLK), jnp.int32),
    )(table, wt_pad, b)
    return out.reshape(VOCAB)


def _sc_body(ptab_hbm, text_hbm, out_hbm, tab_v, idx_v, out_v, sem1, sem2):
    wid = lax.axis_index("s") * NC + lax.axis_index("c")
    base = wid * BPT
    cp1 = pltpu.async_copy(ptab_hbm, tab_v, sem1)
    cp2 = pltpu.async_copy(text_hbm.at[:, pl.ds(base, BPT)], idx_v, sem2)
    cp1.wait()
    cp2.wait()
    scale = jnp.float32(1.0 / SEQ)
    for bg in range(BPT // LANES):
        def body(s, acc, _bg=bg):
            a0, a1 = acc
            vocab = idx_v[s, pl.ds(_bg * LANES, LANES)]
            packed = plsc.load_gather(tab_v, [vocab])
            c0 = plsc.bitcast(packed << 16, jnp.float32)
            c1 = plsc.bitcast(packed & jnp.int32(-65536), jnp.float32)
            return (a0 + c0, a1 + c1)
        z = jnp.zeros((LANES,), jnp.float32)
        a0, a1 = lax.fori_loop(0, SEQ, body, (z, z), unroll=8)
        out_v[0, pl.ds(bg * LANES, LANES)] = a0 * scale
        out_v[1, pl.ds(bg * LANES, LANES)] = a1 * scale
    pltpu.sync_copy(out_v, out_hbm.at[:, pl.ds(base, BPT)])


@functools.lru_cache(maxsize=1)
def _sc_pool():
    return pl.kernel(
        _sc_body,
        out_type=jax.ShapeDtypeStruct((OUT_DIM, BATCH), jnp.float32),
        mesh=plsc.VectorSubcoreMesh(
            core_axis_name="c", subcore_axis_name="s", num_cores=NC, num_subcores=NS
        ),
        scratch_types=[
            pltpu.VMEM((VOCAB,), jnp.int32),
            pltpu.VMEM((SEQ, BPT), jnp.int32),
            pltpu.VMEM((OUT_DIM, BPT), jnp.float32),
            pltpu.SemaphoreType.DMA,
            pltpu.SemaphoreType.DMA,
        ],
        compiler_params=pltpu.CompilerParams(needs_layout_passes=False),
    )


def kernel(text, table, W, b):
    wt_pad = jnp.zeros((8, EMBED), jnp.float32).at[:OUT_DIM].set(W.T)
    ptab = _fold_table(table, wt_pad, b)
    return _sc_pool()(ptab, text).T
